# baseline (speedup 1.0000x reference)
import functools

import jax
import jax.numpy as jnp
from jax import lax
from jax.experimental import pallas as pl
from jax.experimental.pallas import tpu as pltpu
from jax.experimental.pallas import tpu_sc as plsc

_CHUNK_ROWS = 256
_NBUF = 2


def kernel(x, src, dim, start, end, step, out):
    m, d = x.shape
    s = src.shape[0]
    info = plsc.get_sparse_core_info()
    nc = info.num_cores
    ns = info.num_subcores
    nw = nc * ns
    ch = _CHUNK_ROWS
    nb = _NBUF
    src_w = s // nw
    tail_w = (m - s) // nw
    assert s % (nw * ch) == 0 and (m - s) % (nw * ch) == 0
    mesh = plsc.VectorSubcoreMesh(core_axis_name="c", subcore_axis_name="s")

    @functools.partial(
        pl.kernel,
        mesh=mesh,
        out_type=jax.ShapeDtypeStruct((m, d), x.dtype),
        scratch_types=(
            [pltpu.VMEM((ch, d), x.dtype)] * nb
            + [pltpu.VMEM_SHARED((ns * nb, ch, d), x.dtype)]
            + [pltpu.SemaphoreType.DMA] * (4 * nb)
        ),
    )
    def run(x_hbm, src_hbm, out_hbm, *scratch):
        tbufs = scratch[:nb]
        shared = scratch[nb]
        sems = scratch[nb + 1 :]
        cid = lax.axis_index("c")
        sid = lax.axis_index("s")
        wid = sid * nc + cid
        src_base = wid * src_w
        tail_base = s + wid * tail_w

        jobs = [(src_hbm, src_base + i * ch) for i in range(src_w // ch)]
        jobs += [(x_hbm, tail_base + i * ch) for i in range(tail_w // ch)]

        def make_path(pjobs, bufs, sems_r, sems_w):
            def rd(i):
                ref, off = pjobs[i]
                return pltpu.make_async_copy(
                    ref.at[pl.ds(off, ch)], bufs[i % nb], sems_r[i % nb]
                )

            def wr(i):
                off = pjobs[i][1]
                return pltpu.make_async_copy(
                    bufs[i % nb], out_hbm.at[pl.ds(off, ch)], sems_w[i % nb]
                )

            return rd, wr, len(pjobs)

        paths = [
            make_path(jobs[0::2], tbufs, sems[0:nb], sems[nb : 2 * nb]),
            make_path(
                jobs[1::2],
                [shared.at[sid * nb + b] for b in range(nb)],
                sems[2 * nb : 3 * nb],
                sems[3 * nb : 4 * nb],
            ),
        ]

        for rd, wr, pn in paths:
            for i in range(min(nb - 1, pn)):
                rd(i).start()
        maxn = max(pn for _, _, pn in paths)
        for i in range(maxn):
            for rd, wr, pn in paths:
                if i >= pn:
                    continue
                if i + nb - 1 < pn:
                    if i >= 1:
                        wr(i - 1).wait()
                    rd(i + nb - 1).start()
                rd(i).wait()
                wr(i).start()
        for rd, wr, pn in paths:
            for i in range(max(0, pn - nb), pn):
                wr(i).wait()

    return run(x, src)
